# baseline (speedup 1.0000x reference)
import functools
import jax
import jax.numpy as jnp
from jax import lax
from jax.experimental import pallas as pl
from jax.experimental.pallas import tpu as pltpu
from jax.experimental.pallas import tpu_sc as plsc

N_NODES = 10000
E_EDGES = 320000
DIM = 128

NUM_CORES = 2
NUM_SUBCORES = 16
NUM_WORKERS = NUM_CORES * NUM_SUBCORES
CHUNK = 80
NCHUNK = 126
EDGES_PER_W = NCHUNK * CHUNK
N_PAD = 10240
TRASH_ROW = N_NODES
ROWS_PER_SUB = N_PAD // NUM_SUBCORES


def _sc_aggregate(x, src_r, dst_r):
    mesh = plsc.VectorSubcoreMesh(core_axis_name="c", subcore_axis_name="s")

    @functools.partial(
        pl.kernel,
        mesh=mesh,
        out_type=jax.ShapeDtypeStruct((NUM_CORES, N_PAD, DIM), jnp.float32),
        scratch_types=[
            pltpu.VMEM((EDGES_PER_W,), jnp.int32),
            pltpu.VMEM((NCHUNK, CHUNK), jnp.int32),
            pltpu.VMEM((CHUNK, DIM), jnp.float32),
            pltpu.VMEM((CHUNK, DIM), jnp.float32),
            pltpu.VMEM_SHARED((N_PAD, DIM), jnp.float32),
            pltpu.SemaphoreType.DMA,
            pltpu.SemaphoreType.DMA,
            pltpu.SemaphoreType.DMA,
            pltpu.SemaphoreType.DMA,
        ],
    )
    def sc_kernel(x_hbm, src_hbm, dst_hbm, out_hbm,
                  src_v, dst_v, rows0, rows1, aggr_sh, gs0, gs1, ss0, ss1):
        c = lax.axis_index("c")
        s = lax.axis_index("s")
        wid = c * NUM_SUBCORES + s

        idx_cp0 = pltpu.async_copy(src_hbm.at[wid], src_v, gs0)
        idx_cp1 = pltpu.async_copy(dst_hbm.at[wid], dst_v, gs1)

        zeros16 = jnp.zeros((16,), jnp.float32)

        def zbody(i, carry):
            rows0[i // 8, pl.ds((i % 8) * 16, 16)] = zeros16
            return carry

        lax.fori_loop(0, CHUNK * 8, zbody, 0, unroll=8)

        base = s * ROWS_PER_SUB
        for r in range(ROWS_PER_SUB // CHUNK):
            pltpu.sync_copy(rows0,
                            aggr_sh.at[pl.ds(base + r * CHUNK, CHUNK)])
        idx_cp0.wait()
        idx_cp1.wait()
        plsc.subcore_barrier()

        def gath(j, buf, sem):
            return pltpu.make_async_copy(
                x_hbm.at[src_v.at[pl.ds(j * CHUNK, CHUNK)]], buf, sem)

        def scat(j, buf, sem):
            return pltpu.make_async_copy(buf, aggr_sh.at[dst_v.at[j]], sem)

        gath(0, rows0, gs0).start()
        gath(1, rows1, gs1).start()

        def body(i, carry):
            j = 2 * i
            gath(j, rows0, gs0).wait()
            scat(j, rows0, ss0).start(add=True)
            gath(j + 1, rows1, gs1).wait()
            scat(j + 1, rows1, ss1).start(add=True)
            scat(j, rows0, ss0).wait()
            gath(j + 2, rows0, gs0).start()
            scat(j + 1, rows1, ss1).wait()
            gath(j + 3, rows1, gs1).start()
            return carry

        lax.fori_loop(0, NCHUNK // 2 - 1, body, 0)

        j = NCHUNK - 2
        gath(j, rows0, gs0).wait()
        scat(j, rows0, ss0).start(add=True)
        gath(j + 1, rows1, gs1).wait()
        scat(j + 1, rows1, ss1).start(add=True)
        scat(j, rows0, ss0).wait()
        scat(j + 1, rows1, ss1).wait()
        plsc.subcore_barrier()

        pltpu.sync_copy(
            aggr_sh.at[pl.ds(base, ROWS_PER_SUB)],
            out_hbm.at[c, pl.ds(base, ROWS_PER_SUB)],
        )

    return sc_kernel(x, src_r, dst_r)


def _tc_tail(partials, x, W_l, b_l, W_r):
    def tc_kernel(p_ref, x_ref, wl_ref, wr_ref, bl_ref, o_ref):
        aggr = p_ref[0, :N_NODES, :] + p_ref[1, :N_NODES, :]
        h = lax.dot_general(
            aggr, wl_ref[...], (((1,), (1,)), ((), ())),
            preferred_element_type=jnp.float32,
        )
        h = h + lax.dot_general(
            x_ref[...], wr_ref[...], (((1,), (1,)), ((), ())),
            preferred_element_type=jnp.float32,
        )
        o_ref[...] = jnp.maximum(h + bl_ref[...], 0.0)

    return pl.pallas_call(
        tc_kernel,
        out_shape=jax.ShapeDtypeStruct((N_NODES, DIM), jnp.float32),
    )(partials, x, W_l, W_r, b_l.reshape(1, DIM))


@jax.jit
def kernel(x, edge_index, W_l, b_l, W_r):
    pad_w = EDGES_PER_W - E_EDGES // NUM_WORKERS
    src_r = jnp.concatenate(
        [edge_index[0].reshape(NUM_WORKERS, -1),
         jnp.zeros((NUM_WORKERS, pad_w), jnp.int32)], axis=1)
    dst_r = jnp.concatenate(
        [edge_index[1].reshape(NUM_WORKERS, -1),
         jnp.full((NUM_WORKERS, pad_w), TRASH_ROW, jnp.int32)],
        axis=1).reshape(NUM_WORKERS, NCHUNK, CHUNK)
    partials = _sc_aggregate(x, src_r, dst_r)
    return _tc_tail(partials, x, W_l, b_l, W_r)


# speedup vs baseline: 1.4505x; 1.4505x over previous
import functools
import jax
import jax.numpy as jnp
from jax import lax
from jax.experimental import pallas as pl
from jax.experimental.pallas import tpu as pltpu
from jax.experimental.pallas import tpu_sc as plsc

N_NODES = 10000
E_EDGES = 320000
DIM = 128

NUM_CORES = 2
NUM_SUBCORES = 16
NUM_WORKERS = NUM_CORES * NUM_SUBCORES
EDGES_PER_W = E_EDGES // NUM_WORKERS
CHUNK = 80
NCHUNK = EDGES_PER_W // CHUNK
N_PAD = 10240
ROWS_PER_SUB = N_PAD // NUM_SUBCORES


def _sc_aggregate(x, src_r, dst_r):
    mesh = plsc.VectorSubcoreMesh(core_axis_name="c", subcore_axis_name="s")

    @functools.partial(
        pl.kernel,
        mesh=mesh,
        out_type=jax.ShapeDtypeStruct((NUM_CORES, N_PAD, DIM), jnp.float32),
        scratch_types=[
            pltpu.VMEM((EDGES_PER_W,), jnp.int32),
            pltpu.VMEM((NCHUNK, CHUNK), jnp.int32),
            pltpu.VMEM((CHUNK, DIM), jnp.float32),
            pltpu.VMEM((CHUNK, DIM), jnp.float32),
            pltpu.VMEM_SHARED((N_PAD, DIM), jnp.float32),
            pltpu.SemaphoreType.DMA,
            pltpu.SemaphoreType.DMA,
            pltpu.SemaphoreType.DMA,
        ],
    )
    def sc_kernel(x_hbm, src_hbm, dst_hbm, out_hbm,
                  src_v, dst_v, rows0, rows1, aggr_sh, sem0, sem1, ssem):
        c = lax.axis_index("c")
        s = lax.axis_index("s")
        wid = c * NUM_SUBCORES + s

        idx_cp0 = pltpu.async_copy(src_hbm.at[wid], src_v, sem0)
        idx_cp1 = pltpu.async_copy(dst_hbm.at[wid], dst_v, sem1)

        zeros16 = jnp.zeros((16,), jnp.float32)

        def zbody(i, carry):
            rows0[i // 8, pl.ds((i % 8) * 16, 16)] = zeros16
            return carry

        lax.fori_loop(0, CHUNK * 8, zbody, 0, unroll=8)

        base = s * ROWS_PER_SUB
        for r in range(ROWS_PER_SUB // CHUNK):
            pltpu.sync_copy(rows0,
                            aggr_sh.at[pl.ds(base + r * CHUNK, CHUNK)])
        idx_cp0.wait()
        idx_cp1.wait()
        plsc.subcore_barrier()

        def gref(j):
            return x_hbm.at[src_v.at[pl.ds(j * CHUNK, CHUNK)]]

        pltpu.async_copy(gref(0), rows0, sem0)

        def body(i, carry):
            j = 2 * i
            pltpu.async_copy(gref(j + 1), rows1, sem1)
            pltpu.make_async_copy(gref(j), rows0, sem0).wait()
            sc0 = pltpu.make_async_copy(rows0, aggr_sh.at[dst_v.at[j]], ssem)
            sc0.start(add=True)
            pltpu.make_async_copy(gref(j + 1), rows1, sem1).wait()
            pltpu.sync_copy(rows1, aggr_sh.at[dst_v.at[j + 1]], add=True)
            sc0.wait()
            pltpu.async_copy(gref(j + 2), rows0, sem0)
            return carry

        lax.fori_loop(0, (NCHUNK - 1) // 2, body, 0)

        pltpu.make_async_copy(gref(NCHUNK - 1), rows0, sem0).wait()
        pltpu.sync_copy(rows0, aggr_sh.at[dst_v.at[NCHUNK - 1]], add=True)
        plsc.subcore_barrier()

        pltpu.sync_copy(
            aggr_sh.at[pl.ds(base, ROWS_PER_SUB)],
            out_hbm.at[c, pl.ds(base, ROWS_PER_SUB)],
        )

    return sc_kernel(x, src_r, dst_r)


def _tc_tail(partials, x, W_l, b_l, W_r):
    def tc_kernel(p_ref, x_ref, wl_ref, wr_ref, bl_ref, o_ref):
        aggr = p_ref[0, :N_NODES, :] + p_ref[1, :N_NODES, :]
        h = lax.dot_general(
            aggr, wl_ref[...], (((1,), (1,)), ((), ())),
            preferred_element_type=jnp.float32,
        )
        h = h + lax.dot_general(
            x_ref[...], wr_ref[...], (((1,), (1,)), ((), ())),
            preferred_element_type=jnp.float32,
        )
        o_ref[...] = jnp.maximum(h + bl_ref[...], 0.0)

    return pl.pallas_call(
        tc_kernel,
        out_shape=jax.ShapeDtypeStruct((N_NODES, DIM), jnp.float32),
    )(partials, x, W_l, W_r, b_l.reshape(1, DIM))


@jax.jit
def kernel(x, edge_index, W_l, b_l, W_r):
    src_r = edge_index[0].reshape(NUM_WORKERS, EDGES_PER_W)
    dst_r = edge_index[1].reshape(NUM_WORKERS, NCHUNK, CHUNK)
    partials = _sc_aggregate(x, src_r, dst_r)
    return _tc_tail(partials, x, W_l, b_l, W_r)
